# baseline (speedup 1.0000x reference)
import functools

import jax
import jax.numpy as jnp
from jax import lax
from jax.experimental import pallas as pl
from jax.experimental.pallas import tpu as pltpu
from jax.experimental.pallas import tpu_sc as plsc

N = 10000
E = 320000
D = 128
TEMP = 0.5

NC, NS, L = 2, 16, 16
NW = NC * NS
NB = 80
E_PAD = NW * NB * 128
N_ACC = 10240
RPT = N_ACC // NS
EPW = E_PAD // NW


def _make_scatter(Dw, C):
    NBC = NB // C
    mesh = plsc.VectorSubcoreMesh(core_axis_name="c", subcore_axis_name="s")

    @functools.partial(
        pl.kernel,
        mesh=mesh,
        compiler_params=pltpu.CompilerParams(use_tc_tiling_on_sc=False),
        out_type=jax.ShapeDtypeStruct((NC, N_ACC, Dw), jnp.float32),
        scratch_types=[
            pltpu.VMEM((NB, 128), jnp.int32),
            pltpu.VMEM((NB, 128), jnp.int32),
            pltpu.VMEM((C, 128, Dw), jnp.float32),
            pltpu.VMEM_SHARED((N_ACC, Dw), jnp.float32),
            pltpu.SemaphoreType.DMA,
            pltpu.SemaphoreType.DMA,
        ],
    )
    def k(src_hbm, dst_hbm, table_hbm, zrow_hbm, out_hbm,
          src_v, dst_v, rows_v, acc, sem_g, sem_s):
        cid = lax.axis_index("c")
        sid = lax.axis_index("s")
        wid = sid * NC + cid
        pltpu.sync_copy(zrow_hbm, acc.at[pl.ds(sid * RPT, RPT)])
        pltpu.sync_copy(src_hbm.at[wid], src_v)
        pltpu.sync_copy(dst_hbm.at[wid], dst_v)
        plsc.subcore_barrier()

        def chunk(i, carry):
            gh = [pltpu.async_copy(table_hbm.at[src_v.at[i * C + b]],
                                   rows_v.at[b], sem_g)
                  for b in range(C)]
            sh = []
            for b in range(C):
                gh[b].wait()
                sh.append(pltpu.async_copy(rows_v.at[b],
                                           acc.at[dst_v.at[i * C + b]],
                                           sem_s, add=True))
            for b in range(C):
                sh[b].wait()
            return carry

        lax.fori_loop(0, NBC, chunk, 0)
        plsc.subcore_barrier()
        pltpu.sync_copy(acc.at[pl.ds(sid * RPT, RPT)],
                        out_hbm.at[cid, pl.ds(sid * RPT, RPT)])

    return k


_scatter8 = _make_scatter(8, 8)

CS = 5
_mesh_split = plsc.VectorSubcoreMesh(core_axis_name="c", subcore_axis_name="s")


@functools.partial(
    pl.kernel,
    mesh=_mesh_split,
    compiler_params=pltpu.CompilerParams(use_tc_tiling_on_sc=False,
                                         needs_layout_passes=False),
    out_type=[
        jax.ShapeDtypeStruct((NC, 80, 128), jnp.float32),
        jax.ShapeDtypeStruct((NW, EPW), jnp.int32),
        jax.ShapeDtypeStruct((NW, EPW), jnp.int32),
        jax.ShapeDtypeStruct((NW, 16), jnp.int32),
    ],
    scratch_types=[
        pltpu.VMEM((EPW,), jnp.int32),
        pltpu.VMEM((EPW,), jnp.int32),
        pltpu.VMEM((N_ACC,), jnp.float32),
        pltpu.VMEM((N_ACC,), jnp.float32),
        pltpu.VMEM((EPW,), jnp.int32),
        pltpu.VMEM((EPW,), jnp.int32),
        pltpu.VMEM((80, 128), jnp.float32),
        pltpu.VMEM((1, 80), jnp.int32),
        pltpu.VMEM((16,), jnp.int32),
        pltpu.VMEM_SHARED((80, 128), jnp.float32),
    ],
)
def _gate_compact(src_hbm, dst_hbm, inval_hbm, outval_hbm, z80_hbm,
                  sout_hbm, csrc_hbm, cdst_hbm, cnts_hbm,
                  src_v, dst_v, inv_v, outv_v, csrc_v, cdst_v,
                  souts_v, idx80_v, cnt_v, sacc):
    cid = lax.axis_index("c")
    sid = lax.axis_index("s")
    wid = sid * NC + cid
    pltpu.sync_copy(src_hbm.at[wid], src_v)
    pltpu.sync_copy(dst_hbm.at[wid], dst_v)
    pltpu.sync_copy(inval_hbm, inv_v)
    pltpu.sync_copy(outval_hbm, outv_v)
    pltpu.sync_copy(z80_hbm, souts_v)
    pltpu.sync_copy(z80_hbm.at[pl.ds(sid * 5, 5)], sacc.at[pl.ds(sid * 5, 5)])
    for kk in range(5):
        idx80_v[0, pl.ds(kk * 16, 16)] = (
            jnp.arange(16, dtype=jnp.int32) + 16 * kk)

    def pf(i, c):
        csrc_v[pl.ds(i * 16, 16)] = jnp.zeros((16,), jnp.int32)
        idx16 = jnp.arange(16, dtype=jnp.int32) + i * 16
        cdst_v[pl.ds(i * 16, 16)] = N + lax.rem(idx16, N_ACC - N)
        return c

    lax.fori_loop(0, EPW // 16, pf, 0)

    def body(i, off):
        s16 = src_v[pl.ds(i * 16, 16)]
        d16 = dst_v[pl.ds(i * 16, 16)]
        ov = plsc.load_gather(outv_v, [s16])
        iv = plsc.load_gather(inv_v, [d16])
        rr = jnp.right_shift(d16, 7)
        cc = jnp.bitwise_and(d16, 127)
        plsc.addupdate_scatter(souts_v, [rr, cc], ov)
        m = jnp.logical_and(ov != 0.0, iv != 0.0)
        plsc.store_compressed(csrc_v.at[pl.ds(off, 16)], s16, mask=m)
        plsc.store_compressed(cdst_v.at[pl.ds(off, 16)], d16, mask=m)
        return off + plsc.all_reduce_population_count(m)[0]

    total = lax.fori_loop(0, EPW // 16, body, 0)
    plsc.subcore_barrier()
    pltpu.sync_copy(souts_v, sacc.at[idx80_v.at[0]], add=True)
    plsc.subcore_barrier()
    pltpu.sync_copy(sacc.at[pl.ds(sid * 5, 5)],
                    sout_hbm.at[cid, pl.ds(sid * 5, 5)])
    pltpu.sync_copy(csrc_v, csrc_hbm.at[wid])
    pltpu.sync_copy(cdst_v, cdst_hbm.at[wid])
    cnt_v[...] = jnp.broadcast_to(total, (16,)).astype(jnp.int32)
    pltpu.sync_copy(cnt_v, cnts_hbm.at[wid])


@functools.partial(
    pl.kernel,
    mesh=_mesh_split,
    compiler_params=pltpu.CompilerParams(use_tc_tiling_on_sc=False),
    out_type=jax.ShapeDtypeStruct((NC, N_ACC, 64), jnp.float32),
    scratch_types=[
        pltpu.VMEM((NB, 128), jnp.int32),
        pltpu.VMEM((NB, 128), jnp.int32),
        pltpu.VMEM((CS, 128, 64), jnp.float32),
        pltpu.VMEM((16,), jnp.int32),
        pltpu.VMEM_SHARED((N_ACC, 64), jnp.float32),
        pltpu.SemaphoreType.DMA,
        pltpu.SemaphoreType.DMA,
    ],
)
def _scatter_split(csrc_hbm, cdst_hbm, cnts_hbm, table_hbm, zrow_hbm, out_hbm,
                   src_v, dst_v, rows_v, cnt_v, acc, sem_g, sem_s):
    cid = lax.axis_index("c")
    sid = lax.axis_index("s")
    pltpu.sync_copy(zrow_hbm, acc.at[pl.ds(sid * RPT, RPT)])
    plsc.subcore_barrier()

    for g in range(2):
        seg = sid * 2 + g
        pltpu.sync_copy(cnts_hbm.at[seg], cnt_v)
        pltpu.sync_copy(csrc_hbm.at[seg], src_v)
        pltpu.sync_copy(cdst_hbm.at[seg], dst_v)
        nch = (cnt_v[...][0] + (128 * CS - 1)) // (128 * CS)

        def chunk(i, carry):
            @pl.when(i < nch)
            def _():
                gh = [pltpu.async_copy(
                          table_hbm.at[cid].at[src_v.at[i * CS + b]],
                          rows_v.at[b], sem_g)
                      for b in range(CS)]
                sh = []
                for b in range(CS):
                    gh[b].wait()
                    sh.append(pltpu.async_copy(rows_v.at[b],
                                               acc.at[dst_v.at[i * CS + b]],
                                               sem_s, add=True))
                for b in range(CS):
                    sh[b].wait()
            return carry

        lax.fori_loop(0, NB // CS, chunk, 0)

    plsc.subcore_barrier()
    pltpu.sync_copy(acc.at[pl.ds(sid * RPT, RPT)],
                    out_hbm.at[cid, pl.ds(sid * RPT, RPT)])


def _front_body(x_ref, g_ref, b_ref, w4_ref, wenv_ref, h4_ref, henv_ref):
    x = x_ref[...]
    mu = jnp.mean(x, axis=-1, keepdims=True)
    var = jnp.mean((x - mu) ** 2, axis=-1, keepdims=True)
    xn = (x - mu) / jnp.sqrt(var + 1e-5) * g_ref[...] + b_ref[...]
    h4_ref[...] = jnp.dot(xn, w4_ref[...], preferred_element_type=jnp.float32)
    henv_ref[...] = jnp.dot(xn, wenv_ref[...], preferred_element_type=jnp.float32)


def _front(x, ln_g, ln_b, W4, W_env, bs=2000):
    grid = (N // bs,)
    return pl.pallas_call(
        _front_body,
        grid=grid,
        in_specs=[
            pl.BlockSpec((bs, D), lambda i: (i, 0)),
            pl.BlockSpec((1, D), lambda i: (0, 0)),
            pl.BlockSpec((1, D), lambda i: (0, 0)),
            pl.BlockSpec((D, 4), lambda i: (0, 0)),
            pl.BlockSpec((D, D), lambda i: (0, 0)),
        ],
        out_specs=[
            pl.BlockSpec((bs, 4), lambda i: (i, 0)),
            pl.BlockSpec((bs, D), lambda i: (i, 0)),
        ],
        out_shape=[
            jax.ShapeDtypeStruct((N, 4), jnp.float32),
            jax.ShapeDtypeStruct((N, D), jnp.float32),
        ],
    )(x, ln_g.reshape(1, D), ln_b.reshape(1, D), W4, W_env)


def _scale_body(a_ref, h_ref, o_ref):
    hh = a_ref[...] * h_ref[...]
    o_ref[0] = hh[:, :64]
    o_ref[1] = hh[:, 64:]


def _scale_rows_split(a, h, bs=2000):
    return pl.pallas_call(
        _scale_body,
        grid=(N // bs,),
        in_specs=[
            pl.BlockSpec((bs, 1), lambda i: (i, 0)),
            pl.BlockSpec((bs, D), lambda i: (i, 0)),
        ],
        out_specs=pl.BlockSpec((NC, bs, 64), lambda i: (0, i, 0)),
        out_shape=jax.ShapeDtypeStruct((NC, N, 64), jnp.float32),
    )(a.reshape(N, 1), h)


def _final_body(p0_ref, p1_ref, henv_ref, c1_ref, c2_ref, be_ref,
                g_ref, b_ref, o_ref):
    pre = jnp.concatenate([p0_ref[...], p1_ref[...]], axis=1)
    o = (c1_ref[...] * pre
         + c2_ref[...] * henv_ref[...] + be_ref[...])
    mu = jnp.mean(o, axis=-1, keepdims=True)
    var = jnp.mean((o - mu) ** 2, axis=-1, keepdims=True)
    o_ref[...] = (o - mu) / jnp.sqrt(var + 1e-5) * g_ref[...] + b_ref[...]


def _final(p0, p1, h_env, c1, c2, b_env, ln_g, ln_b, bs=2000):
    return pl.pallas_call(
        _final_body,
        grid=(N // bs,),
        in_specs=[
            pl.BlockSpec((bs, 64), lambda i: (i, 0)),
            pl.BlockSpec((bs, 64), lambda i: (i, 0)),
            pl.BlockSpec((bs, D), lambda i: (i, 0)),
            pl.BlockSpec((bs, 1), lambda i: (i, 0)),
            pl.BlockSpec((bs, 1), lambda i: (i, 0)),
            pl.BlockSpec((1, D), lambda i: (0, 0)),
            pl.BlockSpec((1, D), lambda i: (0, 0)),
            pl.BlockSpec((1, D), lambda i: (0, 0)),
        ],
        out_specs=pl.BlockSpec((bs, D), lambda i: (i, 0)),
        out_shape=jax.ShapeDtypeStruct((N, D), jnp.float32),
    )(p0, p1, h_env, c1.reshape(N, 1), c2.reshape(N, 1),
      b_env.reshape(1, D), ln_g.reshape(1, D), ln_b.reshape(1, D))


def _gumbel_hard0(logits, g):
    y = jax.nn.softmax((logits + g) / TEMP, axis=-1)
    idx = jnp.argmax(y, axis=-1)
    y_hard = jax.nn.one_hot(idx, 2, dtype=y.dtype)
    return ((y_hard - y) + y)[:, 0]


def kernel(x, edge_index, W_in, b_in, W_out, b_out, W_env, b_env,
           ln_in_g, ln_in_b, ln_out_g, ln_out_b):
    src, dst = edge_index[0], edge_index[1]
    pad = jnp.zeros((E_PAD - E,), dtype=jnp.int32)
    pad_d = N + jnp.arange(E_PAD - E, dtype=jnp.int32) % (N_ACC - N)
    src3 = jnp.concatenate([src, pad]).reshape(NW, NB, 128)
    dst3 = jnp.concatenate([dst, pad_d]).reshape(NW, NB, 128)

    W4 = jnp.concatenate([W_in, W_out], axis=1)
    b4 = jnp.concatenate([b_in, b_out])
    h4, h_env = _front(x, ln_in_g, ln_in_b, W4, W_env)

    zrow8 = jnp.zeros((RPT, 8), jnp.float32)

    def to8(t):
        return jnp.concatenate(
            [t, jnp.zeros((N, 8 - t.shape[1]), jnp.float32)], axis=1)

    ones_t = to8(jnp.ones((N, 1), jnp.float32))
    cnt = _scatter8(src3, dst3, ones_t, zrow8)
    cnt = cnt[0, :N, 0] + cnt[1, :N, 0]
    dinv_u = 1.0 / jnp.sqrt(cnt + 1.0)

    h4s = to8(dinv_u[:, None] * h4)
    pre4 = _scatter8(src3, dst3, h4s, zrow8)
    pre4 = pre4[0, :N, :4] + pre4[1, :N, :4]
    logits4 = dinv_u[:, None] * pre4 + (dinv_u ** 2)[:, None] * h4 + b4

    kg = jax.random.key(42)
    u1 = jax.random.uniform(jax.random.fold_in(kg, 0), (N, 2),
                            minval=1e-6, maxval=1.0)
    u2 = jax.random.uniform(jax.random.fold_in(kg, 1), (N, 2),
                            minval=1e-6, maxval=1.0)
    g1 = -jnp.log(-jnp.log(u1))
    g2 = -jnp.log(-jnp.log(u2))
    in_val = _gumbel_hard0(logits4[:, :2], g1)
    out_val = _gumbel_hard0(logits4[:, 2:], g2)

    zpad1 = jnp.zeros((N_ACC - N,), jnp.float32)
    src1w = jnp.concatenate([src, pad]).reshape(NW, EPW)
    dst1w = jnp.concatenate([dst, pad_d]).reshape(NW, EPW)
    z80 = jnp.zeros((80, 128), jnp.float32)
    sout_p, csrc, cdst, cnts = _gate_compact(
        src1w, dst1w,
        jnp.concatenate([in_val, zpad1]),
        jnp.concatenate([out_val, zpad1]), z80)
    s_out = (sout_p[0] + sout_p[1]).reshape(N_ACC)[:N]
    deg_w = in_val * s_out + 1.0
    dinv_w = 1.0 / jnp.sqrt(deg_w)

    hh = _scale_rows_split(out_val * dinv_w, h_env)
    zrow64 = jnp.zeros((RPT, 64), jnp.float32)
    pre = _scatter_split(csrc.reshape(NW, NB, 128),
                         cdst.reshape(NW, NB, 128), cnts, hh, zrow64)

    c1 = dinv_w * in_val
    c2 = dinv_w ** 2
    return _final(pre[0, :N], pre[1, :N], h_env, c1, c2,
                  b_env, ln_out_g, ln_out_b)


# speedup vs baseline: 1.0435x; 1.0435x over previous
import functools

import jax
import jax.numpy as jnp
from jax import lax
from jax.experimental import pallas as pl
from jax.experimental.pallas import tpu as pltpu
from jax.experimental.pallas import tpu_sc as plsc

N = 10000
E = 320000
D = 128
TEMP = 0.5

NC, NS, L = 2, 16, 16
NW = NC * NS
NB = 80
E_PAD = NW * NB * 128
N_ACC = 10240
RPT = N_ACC // NS
EPW = E_PAD // NW


def _make_scatter(Dw, C):
    NBC = NB // C
    mesh = plsc.VectorSubcoreMesh(core_axis_name="c", subcore_axis_name="s")

    @functools.partial(
        pl.kernel,
        mesh=mesh,
        compiler_params=pltpu.CompilerParams(use_tc_tiling_on_sc=False),
        out_type=jax.ShapeDtypeStruct((NC, N_ACC, Dw), jnp.float32),
        scratch_types=[
            pltpu.VMEM((NB, 128), jnp.int32),
            pltpu.VMEM((NB, 128), jnp.int32),
            pltpu.VMEM((C, 128, Dw), jnp.float32),
            pltpu.VMEM_SHARED((N_ACC, Dw), jnp.float32),
            pltpu.SemaphoreType.DMA,
            pltpu.SemaphoreType.DMA,
        ],
    )
    def k(src_hbm, dst_hbm, table_hbm, zrow_hbm, out_hbm,
          src_v, dst_v, rows_v, acc, sem_g, sem_s):
        cid = lax.axis_index("c")
        sid = lax.axis_index("s")
        wid = sid * NC + cid
        pltpu.sync_copy(zrow_hbm, acc.at[pl.ds(sid * RPT, RPT)])
        pltpu.sync_copy(src_hbm.at[wid], src_v)
        pltpu.sync_copy(dst_hbm.at[wid], dst_v)
        plsc.subcore_barrier()

        def chunk(i, carry):
            gh = [pltpu.async_copy(table_hbm.at[src_v.at[i * C + b]],
                                   rows_v.at[b], sem_g)
                  for b in range(C)]
            sh = []
            for b in range(C):
                gh[b].wait()
                sh.append(pltpu.async_copy(rows_v.at[b],
                                           acc.at[dst_v.at[i * C + b]],
                                           sem_s, add=True))
            for b in range(C):
                sh[b].wait()
            return carry

        lax.fori_loop(0, NBC, chunk, 0)
        plsc.subcore_barrier()
        pltpu.sync_copy(acc.at[pl.ds(sid * RPT, RPT)],
                        out_hbm.at[cid, pl.ds(sid * RPT, RPT)])

    return k


_scatter8 = _make_scatter(8, 8)

CS = 5
_mesh_split = plsc.VectorSubcoreMesh(core_axis_name="c", subcore_axis_name="s")


@functools.partial(
    pl.kernel,
    mesh=_mesh_split,
    compiler_params=pltpu.CompilerParams(use_tc_tiling_on_sc=False,
                                         needs_layout_passes=False),
    out_type=[
        jax.ShapeDtypeStruct((NC, 80, 128), jnp.float32),
        jax.ShapeDtypeStruct((NW, EPW), jnp.int32),
        jax.ShapeDtypeStruct((NW, EPW), jnp.int32),
        jax.ShapeDtypeStruct((NW, 16), jnp.int32),
    ],
    scratch_types=[
        pltpu.VMEM((EPW,), jnp.int32),
        pltpu.VMEM((EPW,), jnp.int32),
        pltpu.VMEM((N_ACC,), jnp.float32),
        pltpu.VMEM((N_ACC,), jnp.float32),
        pltpu.VMEM((EPW,), jnp.int32),
        pltpu.VMEM((EPW,), jnp.int32),
        pltpu.VMEM((80, 128), jnp.float32),
        pltpu.VMEM((1, 80), jnp.int32),
        pltpu.VMEM((16,), jnp.int32),
        pltpu.VMEM_SHARED((80, 128), jnp.float32),
    ],
)
def _gate_compact(src_hbm, dst_hbm, inval_hbm, outval_hbm, z80_hbm,
                  sout_hbm, csrc_hbm, cdst_hbm, cnts_hbm,
                  src_v, dst_v, inv_v, outv_v, csrc_v, cdst_v,
                  souts_v, idx80_v, cnt_v, sacc):
    cid = lax.axis_index("c")
    sid = lax.axis_index("s")
    wid = sid * NC + cid
    pltpu.sync_copy(src_hbm.at[wid], src_v)
    pltpu.sync_copy(dst_hbm.at[wid], dst_v)
    pltpu.sync_copy(inval_hbm, inv_v)
    pltpu.sync_copy(outval_hbm, outv_v)
    pltpu.sync_copy(z80_hbm, souts_v)
    pltpu.sync_copy(z80_hbm.at[pl.ds(sid * 5, 5)], sacc.at[pl.ds(sid * 5, 5)])
    for kk in range(5):
        idx80_v[0, pl.ds(kk * 16, 16)] = (
            jnp.arange(16, dtype=jnp.int32) + 16 * kk)

    def pf(i, c):
        csrc_v[pl.ds(i * 16, 16)] = jnp.zeros((16,), jnp.int32)
        idx16 = jnp.arange(16, dtype=jnp.int32) + i * 16
        cdst_v[pl.ds(i * 16, 16)] = N + lax.rem(idx16, N_ACC - N)
        return c

    lax.fori_loop(0, EPW // 16, pf, 0)

    def body(i, off):
        s16 = src_v[pl.ds(i * 16, 16)]
        d16 = dst_v[pl.ds(i * 16, 16)]
        ov = plsc.load_gather(outv_v, [s16])
        iv = plsc.load_gather(inv_v, [d16])
        rr = jnp.right_shift(d16, 7)
        cc = jnp.bitwise_and(d16, 127)
        plsc.addupdate_scatter(souts_v, [rr, cc], ov)
        m = jnp.logical_and(ov != 0.0, iv != 0.0)
        plsc.store_compressed(csrc_v.at[pl.ds(off, 16)], s16, mask=m)
        plsc.store_compressed(cdst_v.at[pl.ds(off, 16)], d16, mask=m)
        return off + plsc.all_reduce_population_count(m)[0]

    total = lax.fori_loop(0, EPW // 16, body, 0)
    plsc.subcore_barrier()
    pltpu.sync_copy(souts_v, sacc.at[idx80_v.at[0]], add=True)
    plsc.subcore_barrier()
    pltpu.sync_copy(sacc.at[pl.ds(sid * 5, 5)],
                    sout_hbm.at[cid, pl.ds(sid * 5, 5)])
    pltpu.sync_copy(csrc_v, csrc_hbm.at[wid])
    pltpu.sync_copy(cdst_v, cdst_hbm.at[wid])
    cnt_v[...] = jnp.broadcast_to(total, (16,)).astype(jnp.int32)
    pltpu.sync_copy(cnt_v, cnts_hbm.at[wid])


@functools.partial(
    pl.kernel,
    mesh=_mesh_split,
    compiler_params=pltpu.CompilerParams(use_tc_tiling_on_sc=False),
    out_type=jax.ShapeDtypeStruct((NC, N_ACC, 64), jnp.float32),
    scratch_types=[
        pltpu.VMEM((NB, 128), jnp.int32),
        pltpu.VMEM((NB, 128), jnp.int32),
        pltpu.VMEM((CS, 128, 64), jnp.float32),
        pltpu.VMEM((16,), jnp.int32),
        pltpu.VMEM_SHARED((N_ACC, 64), jnp.float32),
        pltpu.SemaphoreType.DMA,
        pltpu.SemaphoreType.DMA,
    ],
)
def _scatter_split(csrc_hbm, cdst_hbm, cnts_hbm, table_hbm, zrow_hbm, out_hbm,
                   src_v, dst_v, rows_v, cnt_v, acc, sem_g, sem_s):
    cid = lax.axis_index("c")
    sid = lax.axis_index("s")
    pltpu.sync_copy(zrow_hbm.at[pl.ds(sid * RPT, RPT)],
                    acc.at[pl.ds(sid * RPT, RPT)])
    plsc.subcore_barrier()

    for g in range(2):
        seg = sid * 2 + g
        pltpu.sync_copy(cnts_hbm.at[seg], cnt_v)
        pltpu.sync_copy(csrc_hbm.at[seg], src_v)
        pltpu.sync_copy(cdst_hbm.at[seg], dst_v)
        nch = (cnt_v[...][0] + (128 * CS - 1)) // (128 * CS)

        def chunk(i, carry):
            @pl.when(i < nch)
            def _():
                gh = [pltpu.async_copy(
                          table_hbm.at[cid].at[src_v.at[i * CS + b]],
                          rows_v.at[b], sem_g)
                      for b in range(CS)]
                sh = []
                for b in range(CS):
                    gh[b].wait()
                    sh.append(pltpu.async_copy(rows_v.at[b],
                                               acc.at[dst_v.at[i * CS + b]],
                                               sem_s, add=True))
                for b in range(CS):
                    sh[b].wait()
            return carry

        lax.fori_loop(0, NB // CS, chunk, 0)

    plsc.subcore_barrier()
    pltpu.sync_copy(acc.at[pl.ds(sid * RPT, RPT)],
                    out_hbm.at[cid, pl.ds(sid * RPT, RPT)])


def _front_body(x_ref, g_ref, b_ref, w4_ref, wenv_ref, h4_ref, henv_ref):
    x = x_ref[...]
    mu = jnp.mean(x, axis=-1, keepdims=True)
    var = jnp.mean((x - mu) ** 2, axis=-1, keepdims=True)
    xn = (x - mu) / jnp.sqrt(var + 1e-5) * g_ref[...] + b_ref[...]
    h4_ref[...] = jnp.dot(xn, w4_ref[...], preferred_element_type=jnp.float32)
    henv_ref[...] = jnp.dot(xn, wenv_ref[...], preferred_element_type=jnp.float32)


def _front(x, ln_g, ln_b, W4, W_env, bs=2000):
    grid = (N // bs,)
    return pl.pallas_call(
        _front_body,
        grid=grid,
        in_specs=[
            pl.BlockSpec((bs, D), lambda i: (i, 0)),
            pl.BlockSpec((1, D), lambda i: (0, 0)),
            pl.BlockSpec((1, D), lambda i: (0, 0)),
            pl.BlockSpec((D, 4), lambda i: (0, 0)),
            pl.BlockSpec((D, D), lambda i: (0, 0)),
        ],
        out_specs=[
            pl.BlockSpec((bs, 4), lambda i: (i, 0)),
            pl.BlockSpec((bs, D), lambda i: (i, 0)),
        ],
        out_shape=[
            jax.ShapeDtypeStruct((N, 4), jnp.float32),
            jax.ShapeDtypeStruct((N, D), jnp.float32),
        ],
    )(x, ln_g.reshape(1, D), ln_b.reshape(1, D), W4, W_env)


def _scale_body(a_ref, h_ref, o_ref):
    hh = a_ref[...] * h_ref[...]
    o_ref[0] = hh[:, :64]
    o_ref[1] = hh[:, 64:]


def _scale_rows_split(a, h, bs=2000):
    return pl.pallas_call(
        _scale_body,
        grid=(N // bs,),
        in_specs=[
            pl.BlockSpec((bs, 1), lambda i: (i, 0)),
            pl.BlockSpec((bs, D), lambda i: (i, 0)),
        ],
        out_specs=pl.BlockSpec((NC, bs, 64), lambda i: (0, i, 0)),
        out_shape=jax.ShapeDtypeStruct((NC, N, 64), jnp.float32),
    )(a.reshape(N, 1), h)


def _final_body(p0_ref, p1_ref, henv_ref, c1_ref, c2_ref, be_ref,
                g_ref, b_ref, o_ref):
    pre = jnp.concatenate([p0_ref[...], p1_ref[...]], axis=1)
    o = (c1_ref[...] * pre
         + c2_ref[...] * henv_ref[...] + be_ref[...])
    mu = jnp.mean(o, axis=-1, keepdims=True)
    var = jnp.mean((o - mu) ** 2, axis=-1, keepdims=True)
    o_ref[...] = (o - mu) / jnp.sqrt(var + 1e-5) * g_ref[...] + b_ref[...]


def _final(p0, p1, h_env, c1, c2, b_env, ln_g, ln_b, bs=2000):
    return pl.pallas_call(
        _final_body,
        grid=(N // bs,),
        in_specs=[
            pl.BlockSpec((bs, 64), lambda i: (i, 0)),
            pl.BlockSpec((bs, 64), lambda i: (i, 0)),
            pl.BlockSpec((bs, D), lambda i: (i, 0)),
            pl.BlockSpec((bs, 1), lambda i: (i, 0)),
            pl.BlockSpec((bs, 1), lambda i: (i, 0)),
            pl.BlockSpec((1, D), lambda i: (0, 0)),
            pl.BlockSpec((1, D), lambda i: (0, 0)),
            pl.BlockSpec((1, D), lambda i: (0, 0)),
        ],
        out_specs=pl.BlockSpec((bs, D), lambda i: (i, 0)),
        out_shape=jax.ShapeDtypeStruct((N, D), jnp.float32),
    )(p0, p1, h_env, c1.reshape(N, 1), c2.reshape(N, 1),
      b_env.reshape(1, D), ln_g.reshape(1, D), ln_b.reshape(1, D))


def _gumbel_hard0(logits, g):
    y = jax.nn.softmax((logits + g) / TEMP, axis=-1)
    idx = jnp.argmax(y, axis=-1)
    y_hard = jax.nn.one_hot(idx, 2, dtype=y.dtype)
    return ((y_hard - y) + y)[:, 0]


def kernel(x, edge_index, W_in, b_in, W_out, b_out, W_env, b_env,
           ln_in_g, ln_in_b, ln_out_g, ln_out_b):
    src, dst = edge_index[0], edge_index[1]
    pad = jnp.zeros((E_PAD - E,), dtype=jnp.int32)
    pad_d = N + jnp.arange(E_PAD - E, dtype=jnp.int32) % (N_ACC - N)
    src3 = jnp.concatenate([src, pad]).reshape(NW, NB, 128)
    dst3 = jnp.concatenate([dst, pad_d]).reshape(NW, NB, 128)

    W4 = jnp.concatenate([W_in, W_out], axis=1)
    b4 = jnp.concatenate([b_in, b_out])
    h4, h_env = _front(x, ln_in_g, ln_in_b, W4, W_env)

    zrow8 = jnp.zeros((RPT, 8), jnp.float32)

    def to8(t):
        return jnp.concatenate(
            [t, jnp.zeros((N, 8 - t.shape[1]), jnp.float32)], axis=1)

    ones_t = to8(jnp.ones((N, 1), jnp.float32))
    cnt = _scatter8(src3, dst3, ones_t, zrow8)
    cnt = cnt[0, :N, 0] + cnt[1, :N, 0]
    dinv_u = 1.0 / jnp.sqrt(cnt + 1.0)

    h4s = to8(dinv_u[:, None] * h4)
    pre4 = _scatter8(src3, dst3, h4s, zrow8)
    pre4 = pre4[0, :N, :4] + pre4[1, :N, :4]
    logits4 = dinv_u[:, None] * pre4 + (dinv_u ** 2)[:, None] * h4 + b4

    kg = jax.random.key(42)
    u1 = jax.random.uniform(jax.random.fold_in(kg, 0), (N, 2),
                            minval=1e-6, maxval=1.0)
    u2 = jax.random.uniform(jax.random.fold_in(kg, 1), (N, 2),
                            minval=1e-6, maxval=1.0)
    g1 = -jnp.log(-jnp.log(u1))
    g2 = -jnp.log(-jnp.log(u2))
    in_val = _gumbel_hard0(logits4[:, :2], g1)
    out_val = _gumbel_hard0(logits4[:, 2:], g2)

    zpad1 = jnp.zeros((N_ACC - N,), jnp.float32)
    src1w = jnp.concatenate([src, pad]).reshape(NW, EPW)
    dst1w = jnp.concatenate([dst, pad_d]).reshape(NW, EPW)
    z80 = jnp.zeros((80, 128), jnp.float32)
    sout_p, csrc, cdst, cnts = _gate_compact(
        src1w, dst1w,
        jnp.concatenate([in_val, zpad1]),
        jnp.concatenate([out_val, zpad1]), z80)
    s_out = (sout_p[0] + sout_p[1]).reshape(N_ACC)[:N]
    deg_w = in_val * s_out + 1.0
    dinv_w = 1.0 / jnp.sqrt(deg_w)

    hh = _scale_rows_split(out_val * dinv_w, h_env)
    zrow64 = jnp.zeros((N_ACC, 64), jnp.float32)
    pre = _scatter_split(csrc.reshape(NW, NB, 128),
                         cdst.reshape(NW, NB, 128), cnts, hh, zrow64)

    c1 = dinv_w * in_val
    c2 = dinv_w ** 2
    return _final(pre[0, :N], pre[1, :N], h_env, c1, c2,
                  b_env, ln_out_g, ln_out_b)


# speedup vs baseline: 1.1256x; 1.0788x over previous
import functools

import jax
import jax.numpy as jnp
from jax import lax
from jax.experimental import pallas as pl
from jax.experimental.pallas import tpu as pltpu
from jax.experimental.pallas import tpu_sc as plsc

N = 10000
E = 320000
D = 128
TEMP = 0.5

NC, NS, L = 2, 16, 16
NW = NC * NS
NB = 80
E_PAD = NW * NB * 128
N_ACC = 10240
RPT = N_ACC // NS
EPW = E_PAD // NW


def _make_scatter(Dw, C):
    NBC = NB // C
    mesh = plsc.VectorSubcoreMesh(core_axis_name="c", subcore_axis_name="s")

    @functools.partial(
        pl.kernel,
        mesh=mesh,
        compiler_params=pltpu.CompilerParams(use_tc_tiling_on_sc=False),
        out_type=jax.ShapeDtypeStruct((NC, N_ACC, Dw), jnp.float32),
        scratch_types=[
            pltpu.VMEM((NB, 128), jnp.int32),
            pltpu.VMEM((NB, 128), jnp.int32),
            pltpu.VMEM((C, 128, Dw), jnp.float32),
            pltpu.VMEM_SHARED((N_ACC, Dw), jnp.float32),
            pltpu.SemaphoreType.DMA,
            pltpu.SemaphoreType.DMA,
        ],
    )
    def k(src_hbm, dst_hbm, table_hbm, zrow_hbm, out_hbm,
          src_v, dst_v, rows_v, acc, sem_g, sem_s):
        cid = lax.axis_index("c")
        sid = lax.axis_index("s")
        wid = sid * NC + cid
        pltpu.sync_copy(zrow_hbm, acc.at[pl.ds(sid * RPT, RPT)])
        pltpu.sync_copy(src_hbm.at[wid], src_v)
        pltpu.sync_copy(dst_hbm.at[wid], dst_v)
        plsc.subcore_barrier()

        def chunk(i, carry):
            gh = [pltpu.async_copy(table_hbm.at[src_v.at[i * C + b]],
                                   rows_v.at[b], sem_g)
                  for b in range(C)]
            sh = []
            for b in range(C):
                gh[b].wait()
                sh.append(pltpu.async_copy(rows_v.at[b],
                                           acc.at[dst_v.at[i * C + b]],
                                           sem_s, add=True))
            for b in range(C):
                sh[b].wait()
            return carry

        lax.fori_loop(0, NBC, chunk, 0)
        plsc.subcore_barrier()
        pltpu.sync_copy(acc.at[pl.ds(sid * RPT, RPT)],
                        out_hbm.at[cid, pl.ds(sid * RPT, RPT)])

    return k


_scatter8 = _make_scatter(8, 8)

CS = 5
_mesh_split = plsc.VectorSubcoreMesh(core_axis_name="c", subcore_axis_name="s")


@functools.partial(
    pl.kernel,
    mesh=_mesh_split,
    compiler_params=pltpu.CompilerParams(use_tc_tiling_on_sc=False,
                                         needs_layout_passes=False),
    out_type=[
        jax.ShapeDtypeStruct((NC, 80, 128), jnp.float32),
        jax.ShapeDtypeStruct((NW, EPW), jnp.int32),
        jax.ShapeDtypeStruct((NW, EPW), jnp.int32),
        jax.ShapeDtypeStruct((NW, 16), jnp.int32),
    ],
    scratch_types=[
        pltpu.VMEM((EPW,), jnp.int32),
        pltpu.VMEM((EPW,), jnp.int32),
        pltpu.VMEM((N_ACC,), jnp.float32),
        pltpu.VMEM((N_ACC,), jnp.float32),
        pltpu.VMEM((EPW,), jnp.int32),
        pltpu.VMEM((EPW,), jnp.int32),
        pltpu.VMEM((80, 128), jnp.float32),
        pltpu.VMEM((1, 80), jnp.int32),
        pltpu.VMEM((16,), jnp.int32),
        pltpu.VMEM_SHARED((80, 128), jnp.float32),
    ],
)
def _gate_compact(src_hbm, dst_hbm, inval_hbm, outval_hbm, z80_hbm,
                  sout_hbm, csrc_hbm, cdst_hbm, cnts_hbm,
                  src_v, dst_v, inv_v, outv_v, csrc_v, cdst_v,
                  souts_v, idx80_v, cnt_v, sacc):
    cid = lax.axis_index("c")
    sid = lax.axis_index("s")
    wid = sid * NC + cid
    pltpu.sync_copy(src_hbm.at[wid], src_v)
    pltpu.sync_copy(dst_hbm.at[wid], dst_v)
    pltpu.sync_copy(inval_hbm, inv_v)
    pltpu.sync_copy(outval_hbm, outv_v)
    pltpu.sync_copy(z80_hbm, souts_v)
    pltpu.sync_copy(z80_hbm.at[pl.ds(sid * 5, 5)], sacc.at[pl.ds(sid * 5, 5)])
    for kk in range(5):
        idx80_v[0, pl.ds(kk * 16, 16)] = (
            jnp.arange(16, dtype=jnp.int32) + 16 * kk)

    def pf(i, c):
        csrc_v[pl.ds(i * 16, 16)] = jnp.zeros((16,), jnp.int32)
        idx16 = jnp.arange(16, dtype=jnp.int32) + i * 16
        cdst_v[pl.ds(i * 16, 16)] = N + lax.rem(idx16, N_ACC - N)
        return c

    lax.fori_loop(0, EPW // 16, pf, 0)

    def body(i, off):
        s16 = src_v[pl.ds(i * 16, 16)]
        d16 = dst_v[pl.ds(i * 16, 16)]
        ov = plsc.load_gather(outv_v, [s16])
        iv = plsc.load_gather(inv_v, [d16])
        rr = jnp.right_shift(d16, 7)
        cc = jnp.bitwise_and(d16, 127)
        plsc.addupdate_scatter(souts_v, [rr, cc], ov)
        m = jnp.logical_and(ov != 0.0, iv != 0.0)
        plsc.store_compressed(csrc_v.at[pl.ds(off, 16)], s16, mask=m)
        plsc.store_compressed(cdst_v.at[pl.ds(off, 16)], d16, mask=m)
        return off + plsc.all_reduce_population_count(m)[0]

    total = lax.fori_loop(0, EPW // 16, body, 0)
    plsc.subcore_barrier()
    pltpu.sync_copy(souts_v, sacc.at[idx80_v.at[0]], add=True)
    plsc.subcore_barrier()
    pltpu.sync_copy(sacc.at[pl.ds(sid * 5, 5)],
                    sout_hbm.at[cid, pl.ds(sid * 5, 5)])
    pltpu.sync_copy(csrc_v, csrc_hbm.at[wid])
    pltpu.sync_copy(cdst_v, cdst_hbm.at[wid])
    cnt_v[...] = jnp.broadcast_to(total, (16,)).astype(jnp.int32)
    pltpu.sync_copy(cnt_v, cnts_hbm.at[wid])


NBT = E_PAD // (NS * 128)


@functools.partial(
    pl.kernel,
    mesh=_mesh_split,
    compiler_params=pltpu.CompilerParams(use_tc_tiling_on_sc=False),
    out_type=jax.ShapeDtypeStruct((NC, N_ACC, 64), jnp.float32),
    scratch_types=[
        pltpu.VMEM((NBT, 128), jnp.int32),
        pltpu.VMEM((NBT, 128), jnp.int32),
        pltpu.VMEM((CS, 128, 64), jnp.float32),
        pltpu.VMEM_SHARED((N_ACC, 64), jnp.float32),
        pltpu.SemaphoreType.DMA,
        pltpu.SemaphoreType.DMA,
    ],
)
def _scatter_split(src_hbm, dst_hbm, table_hbm, zrow_hbm, out_hbm,
                   src_v, dst_v, rows_v, acc, sem_g, sem_s):
    cid = lax.axis_index("c")
    sid = lax.axis_index("s")
    pltpu.sync_copy(zrow_hbm.at[pl.ds(sid * RPT, RPT)],
                    acc.at[pl.ds(sid * RPT, RPT)])
    pltpu.sync_copy(src_hbm.at[sid], src_v)
    pltpu.sync_copy(dst_hbm.at[sid], dst_v)
    plsc.subcore_barrier()

    def chunk(i, carry):
        gh = [pltpu.async_copy(table_hbm.at[cid].at[src_v.at[i * CS + b]],
                               rows_v.at[b], sem_g)
              for b in range(CS)]
        sh = []
        for b in range(CS):
            gh[b].wait()
            sh.append(pltpu.async_copy(rows_v.at[b],
                                       acc.at[dst_v.at[i * CS + b]],
                                       sem_s, add=True))
        for b in range(CS):
            sh[b].wait()
        return carry

    lax.fori_loop(0, NBT // CS, chunk, 0)
    plsc.subcore_barrier()
    pltpu.sync_copy(acc.at[pl.ds(sid * RPT, RPT)],
                    out_hbm.at[cid, pl.ds(sid * RPT, RPT)])


def _front_body(x_ref, g_ref, b_ref, w4_ref, wenv_ref, h4_ref, henv_ref):
    x = x_ref[...]
    mu = jnp.mean(x, axis=-1, keepdims=True)
    var = jnp.mean((x - mu) ** 2, axis=-1, keepdims=True)
    xn = (x - mu) / jnp.sqrt(var + 1e-5) * g_ref[...] + b_ref[...]
    h4_ref[...] = jnp.dot(xn, w4_ref[...], preferred_element_type=jnp.float32)
    henv_ref[...] = jnp.dot(xn, wenv_ref[...], preferred_element_type=jnp.float32)


def _front(x, ln_g, ln_b, W4, W_env, bs=2000):
    grid = (N // bs,)
    return pl.pallas_call(
        _front_body,
        grid=grid,
        in_specs=[
            pl.BlockSpec((bs, D), lambda i: (i, 0)),
            pl.BlockSpec((1, D), lambda i: (0, 0)),
            pl.BlockSpec((1, D), lambda i: (0, 0)),
            pl.BlockSpec((D, 4), lambda i: (0, 0)),
            pl.BlockSpec((D, D), lambda i: (0, 0)),
        ],
        out_specs=[
            pl.BlockSpec((bs, 4), lambda i: (i, 0)),
            pl.BlockSpec((bs, D), lambda i: (i, 0)),
        ],
        out_shape=[
            jax.ShapeDtypeStruct((N, 4), jnp.float32),
            jax.ShapeDtypeStruct((N, D), jnp.float32),
        ],
    )(x, ln_g.reshape(1, D), ln_b.reshape(1, D), W4, W_env)


def _scale_body(a_ref, h_ref, o_ref):
    hh = a_ref[...] * h_ref[...]
    o_ref[0] = hh[:, :64]
    o_ref[1] = hh[:, 64:]


def _scale_rows_split(a, h, bs=2000):
    return pl.pallas_call(
        _scale_body,
        grid=(N // bs,),
        in_specs=[
            pl.BlockSpec((bs, 1), lambda i: (i, 0)),
            pl.BlockSpec((bs, D), lambda i: (i, 0)),
        ],
        out_specs=pl.BlockSpec((NC, bs, 64), lambda i: (0, i, 0)),
        out_shape=jax.ShapeDtypeStruct((NC, N, 64), jnp.float32),
    )(a.reshape(N, 1), h)


def _final_body(p0_ref, p1_ref, henv_ref, c1_ref, c2_ref, be_ref,
                g_ref, b_ref, o_ref):
    pre = jnp.concatenate([p0_ref[...], p1_ref[...]], axis=1)
    o = (c1_ref[...] * pre
         + c2_ref[...] * henv_ref[...] + be_ref[...])
    mu = jnp.mean(o, axis=-1, keepdims=True)
    var = jnp.mean((o - mu) ** 2, axis=-1, keepdims=True)
    o_ref[...] = (o - mu) / jnp.sqrt(var + 1e-5) * g_ref[...] + b_ref[...]


def _final(p0, p1, h_env, c1, c2, b_env, ln_g, ln_b, bs=2000):
    return pl.pallas_call(
        _final_body,
        grid=(N // bs,),
        in_specs=[
            pl.BlockSpec((bs, 64), lambda i: (i, 0)),
            pl.BlockSpec((bs, 64), lambda i: (i, 0)),
            pl.BlockSpec((bs, D), lambda i: (i, 0)),
            pl.BlockSpec((bs, 1), lambda i: (i, 0)),
            pl.BlockSpec((bs, 1), lambda i: (i, 0)),
            pl.BlockSpec((1, D), lambda i: (0, 0)),
            pl.BlockSpec((1, D), lambda i: (0, 0)),
            pl.BlockSpec((1, D), lambda i: (0, 0)),
        ],
        out_specs=pl.BlockSpec((bs, D), lambda i: (i, 0)),
        out_shape=jax.ShapeDtypeStruct((N, D), jnp.float32),
    )(p0, p1, h_env, c1.reshape(N, 1), c2.reshape(N, 1),
      b_env.reshape(1, D), ln_g.reshape(1, D), ln_b.reshape(1, D))


def _gumbel_hard0(logits, g):
    y = jax.nn.softmax((logits + g) / TEMP, axis=-1)
    idx = jnp.argmax(y, axis=-1)
    y_hard = jax.nn.one_hot(idx, 2, dtype=y.dtype)
    return ((y_hard - y) + y)[:, 0]


def kernel(x, edge_index, W_in, b_in, W_out, b_out, W_env, b_env,
           ln_in_g, ln_in_b, ln_out_g, ln_out_b):
    src, dst = edge_index[0], edge_index[1]
    pad = jnp.zeros((E_PAD - E,), dtype=jnp.int32)
    pad_d = N + jnp.arange(E_PAD - E, dtype=jnp.int32) % (N_ACC - N)
    src3 = jnp.concatenate([src, pad]).reshape(NW, NB, 128)
    dst3 = jnp.concatenate([dst, pad_d]).reshape(NW, NB, 128)

    W4 = jnp.concatenate([W_in, W_out], axis=1)
    b4 = jnp.concatenate([b_in, b_out])
    h4, h_env = _front(x, ln_in_g, ln_in_b, W4, W_env)

    zrow8 = jnp.zeros((RPT, 8), jnp.float32)

    def to8(t):
        return jnp.concatenate(
            [t, jnp.zeros((N, 8 - t.shape[1]), jnp.float32)], axis=1)

    ones_t = to8(jnp.ones((N, 1), jnp.float32))
    cnt = _scatter8(src3, dst3, ones_t, zrow8)
    cnt = cnt[0, :N, 0] + cnt[1, :N, 0]
    dinv_u = 1.0 / jnp.sqrt(cnt + 1.0)

    h4s = to8(dinv_u[:, None] * h4)
    pre4 = _scatter8(src3, dst3, h4s, zrow8)
    pre4 = pre4[0, :N, :4] + pre4[1, :N, :4]
    logits4 = dinv_u[:, None] * pre4 + (dinv_u ** 2)[:, None] * h4 + b4

    kg = jax.random.key(42)
    u1 = jax.random.uniform(jax.random.fold_in(kg, 0), (N, 2),
                            minval=1e-6, maxval=1.0)
    u2 = jax.random.uniform(jax.random.fold_in(kg, 1), (N, 2),
                            minval=1e-6, maxval=1.0)
    g1 = -jnp.log(-jnp.log(u1))
    g2 = -jnp.log(-jnp.log(u2))
    in_val = _gumbel_hard0(logits4[:, :2], g1)
    out_val = _gumbel_hard0(logits4[:, 2:], g2)

    zpad1 = jnp.zeros((N_ACC - N,), jnp.float32)
    src1w = jnp.concatenate([src, pad]).reshape(NW, EPW)
    dst1w = jnp.concatenate([dst, pad_d]).reshape(NW, EPW)
    z80 = jnp.zeros((80, 128), jnp.float32)
    sout_p, csrc, cdst, cnts = _gate_compact(
        src1w, dst1w,
        jnp.concatenate([in_val, zpad1]),
        jnp.concatenate([out_val, zpad1]), z80)
    s_out = (sout_p[0] + sout_p[1]).reshape(N_ACC)[:N]
    deg_w = in_val * s_out + 1.0
    dinv_w = 1.0 / jnp.sqrt(deg_w)

    hh = _scale_rows_split(out_val * dinv_w, h_env)
    zrow64 = jnp.zeros((N_ACC, 64), jnp.float32)
    src3s = jnp.concatenate([src, pad]).reshape(NS, NBT, 128)
    dst3s = jnp.concatenate([dst, pad_d]).reshape(NS, NBT, 128)
    pre = _scatter_split(src3s, dst3s, hh, zrow64)

    c1 = dinv_w * in_val
    c2 = dinv_w ** 2
    return _final(pre[0, :N], pre[1, :N], h_env, c1, c2,
                  b_env, ln_out_g, ln_out_b)


# speedup vs baseline: 1.1455x; 1.0176x over previous
import functools

import jax
import jax.numpy as jnp
from jax import lax
from jax.experimental import pallas as pl
from jax.experimental.pallas import tpu as pltpu
from jax.experimental.pallas import tpu_sc as plsc

N = 10000
E = 320000
D = 128
TEMP = 0.5

NC, NS, L = 2, 16, 16
NW = NC * NS
NB = 80
E_PAD = NW * NB * 128
N_ACC = 10240
RPT = N_ACC // NS
EPW = E_PAD // NW


def _make_scatter(Dw, C):
    NBC = NB // C
    mesh = plsc.VectorSubcoreMesh(core_axis_name="c", subcore_axis_name="s")

    @functools.partial(
        pl.kernel,
        mesh=mesh,
        compiler_params=pltpu.CompilerParams(use_tc_tiling_on_sc=False),
        out_type=jax.ShapeDtypeStruct((NC, N_ACC, Dw), jnp.float32),
        scratch_types=[
            pltpu.VMEM((NB, 128), jnp.int32),
            pltpu.VMEM((NB, 128), jnp.int32),
            pltpu.VMEM((C, 128, Dw), jnp.float32),
            pltpu.VMEM_SHARED((N_ACC, Dw), jnp.float32),
            pltpu.SemaphoreType.DMA,
            pltpu.SemaphoreType.DMA,
        ],
    )
    def k(src_hbm, dst_hbm, table_hbm, zrow_hbm, out_hbm,
          src_v, dst_v, rows_v, acc, sem_g, sem_s):
        cid = lax.axis_index("c")
        sid = lax.axis_index("s")
        wid = sid * NC + cid
        pltpu.sync_copy(zrow_hbm, acc.at[pl.ds(sid * RPT, RPT)])
        pltpu.sync_copy(src_hbm.at[wid], src_v)
        pltpu.sync_copy(dst_hbm.at[wid], dst_v)
        plsc.subcore_barrier()

        def chunk(i, carry):
            gh = [pltpu.async_copy(table_hbm.at[src_v.at[i * C + b]],
                                   rows_v.at[b], sem_g)
                  for b in range(C)]
            sh = []
            for b in range(C):
                gh[b].wait()
                sh.append(pltpu.async_copy(rows_v.at[b],
                                           acc.at[dst_v.at[i * C + b]],
                                           sem_s, add=True))
            for b in range(C):
                sh[b].wait()
            return carry

        lax.fori_loop(0, NBC, chunk, 0)
        plsc.subcore_barrier()
        pltpu.sync_copy(acc.at[pl.ds(sid * RPT, RPT)],
                        out_hbm.at[cid, pl.ds(sid * RPT, RPT)])

    return k


_scatter8 = _make_scatter(8, 16)

CS = 5
_mesh_split = plsc.VectorSubcoreMesh(core_axis_name="c", subcore_axis_name="s")


@functools.partial(
    pl.kernel,
    mesh=_mesh_split,
    compiler_params=pltpu.CompilerParams(use_tc_tiling_on_sc=False,
                                         needs_layout_passes=False),
    out_type=jax.ShapeDtypeStruct((NC, 80, 128), jnp.float32),
    scratch_types=[
        pltpu.VMEM((EPW,), jnp.int32),
        pltpu.VMEM((EPW,), jnp.int32),
        pltpu.VMEM((N_ACC,), jnp.float32),
        pltpu.VMEM((80, 128), jnp.float32),
        pltpu.VMEM((1, 80), jnp.int32),
        pltpu.VMEM_SHARED((80, 128), jnp.float32),
    ],
)
def _gate_sout(src_hbm, dst_hbm, outval_hbm, z80_hbm, sout_hbm,
               src_v, dst_v, outv_v, souts_v, idx80_v, sacc):
    cid = lax.axis_index("c")
    sid = lax.axis_index("s")
    wid = sid * NC + cid
    pltpu.sync_copy(src_hbm.at[wid], src_v)
    pltpu.sync_copy(dst_hbm.at[wid], dst_v)
    pltpu.sync_copy(outval_hbm, outv_v)
    pltpu.sync_copy(z80_hbm, souts_v)
    pltpu.sync_copy(z80_hbm.at[pl.ds(sid * 5, 5)], sacc.at[pl.ds(sid * 5, 5)])
    for kk in range(5):
        idx80_v[0, pl.ds(kk * 16, 16)] = (
            jnp.arange(16, dtype=jnp.int32) + 16 * kk)

    def body(i, c):
        s16 = src_v[pl.ds(i * 16, 16)]
        d16 = dst_v[pl.ds(i * 16, 16)]
        ov = plsc.load_gather(outv_v, [s16])
        rr = jnp.right_shift(d16, 7)
        cc = jnp.bitwise_and(d16, 127)
        plsc.addupdate_scatter(souts_v, [rr, cc], ov)
        return c

    lax.fori_loop(0, EPW // 16, body, 0)
    plsc.subcore_barrier()
    pltpu.sync_copy(souts_v, sacc.at[idx80_v.at[0]], add=True)
    plsc.subcore_barrier()
    pltpu.sync_copy(sacc.at[pl.ds(sid * 5, 5)],
                    sout_hbm.at[cid, pl.ds(sid * 5, 5)])


NBT = E_PAD // (NS * 128)


@functools.partial(
    pl.kernel,
    mesh=_mesh_split,
    compiler_params=pltpu.CompilerParams(use_tc_tiling_on_sc=False),
    out_type=jax.ShapeDtypeStruct((NC, N_ACC, 64), jnp.float32),
    scratch_types=[
        pltpu.VMEM((NBT, 128), jnp.int32),
        pltpu.VMEM((NBT, 128), jnp.int32),
        pltpu.VMEM((CS, 128, 64), jnp.float32),
        pltpu.VMEM_SHARED((N_ACC, 64), jnp.float32),
        pltpu.SemaphoreType.DMA,
        pltpu.SemaphoreType.DMA,
    ],
)
def _scatter_split(src_hbm, dst_hbm, table_hbm, zrow_hbm, out_hbm,
                   src_v, dst_v, rows_v, acc, sem_g, sem_s):
    cid = lax.axis_index("c")
    sid = lax.axis_index("s")
    pltpu.sync_copy(zrow_hbm.at[pl.ds(sid * RPT, RPT)],
                    acc.at[pl.ds(sid * RPT, RPT)])
    pltpu.sync_copy(src_hbm.at[sid], src_v)
    pltpu.sync_copy(dst_hbm.at[sid], dst_v)
    plsc.subcore_barrier()

    def chunk(i, carry):
        gh = [pltpu.async_copy(table_hbm.at[cid].at[src_v.at[i * CS + b]],
                               rows_v.at[b], sem_g)
              for b in range(CS)]
        sh = []
        for b in range(CS):
            gh[b].wait()
            sh.append(pltpu.async_copy(rows_v.at[b],
                                       acc.at[dst_v.at[i * CS + b]],
                                       sem_s, add=True))
        for b in range(CS):
            sh[b].wait()
        return carry

    lax.fori_loop(0, NBT // CS, chunk, 0)
    plsc.subcore_barrier()
    pltpu.sync_copy(acc.at[pl.ds(sid * RPT, RPT)],
                    out_hbm.at[cid, pl.ds(sid * RPT, RPT)])


def _front_body(x_ref, g_ref, b_ref, w4_ref, wenv_ref, h4_ref, henv_ref):
    x = x_ref[...]
    mu = jnp.mean(x, axis=-1, keepdims=True)
    var = jnp.mean((x - mu) ** 2, axis=-1, keepdims=True)
    xn = (x - mu) / jnp.sqrt(var + 1e-5) * g_ref[...] + b_ref[...]
    h4_ref[...] = jnp.dot(xn, w4_ref[...], preferred_element_type=jnp.float32)
    henv_ref[...] = jnp.dot(xn, wenv_ref[...], preferred_element_type=jnp.float32)


def _front(x, ln_g, ln_b, W4, W_env, bs=2000):
    grid = (N // bs,)
    return pl.pallas_call(
        _front_body,
        grid=grid,
        in_specs=[
            pl.BlockSpec((bs, D), lambda i: (i, 0)),
            pl.BlockSpec((1, D), lambda i: (0, 0)),
            pl.BlockSpec((1, D), lambda i: (0, 0)),
            pl.BlockSpec((D, 4), lambda i: (0, 0)),
            pl.BlockSpec((D, D), lambda i: (0, 0)),
        ],
        out_specs=[
            pl.BlockSpec((bs, 4), lambda i: (i, 0)),
            pl.BlockSpec((bs, D), lambda i: (i, 0)),
        ],
        out_shape=[
            jax.ShapeDtypeStruct((N, 4), jnp.float32),
            jax.ShapeDtypeStruct((N, D), jnp.float32),
        ],
    )(x, ln_g.reshape(1, D), ln_b.reshape(1, D), W4, W_env)


def _scale_body(a_ref, h_ref, o_ref):
    hh = a_ref[...] * h_ref[...]
    o_ref[0] = hh[:, :64]
    o_ref[1] = hh[:, 64:]


def _scale_rows_split(a, h, bs=2000):
    return pl.pallas_call(
        _scale_body,
        grid=(N // bs,),
        in_specs=[
            pl.BlockSpec((bs, 1), lambda i: (i, 0)),
            pl.BlockSpec((bs, D), lambda i: (i, 0)),
        ],
        out_specs=pl.BlockSpec((NC, bs, 64), lambda i: (0, i, 0)),
        out_shape=jax.ShapeDtypeStruct((NC, N, 64), jnp.float32),
    )(a.reshape(N, 1), h)


def _final_body(p0_ref, p1_ref, henv_ref, c1_ref, c2_ref, be_ref,
                g_ref, b_ref, o_ref):
    pre = jnp.concatenate([p0_ref[...], p1_ref[...]], axis=1)
    o = (c1_ref[...] * pre
         + c2_ref[...] * henv_ref[...] + be_ref[...])
    mu = jnp.mean(o, axis=-1, keepdims=True)
    var = jnp.mean((o - mu) ** 2, axis=-1, keepdims=True)
    o_ref[...] = (o - mu) / jnp.sqrt(var + 1e-5) * g_ref[...] + b_ref[...]


def _final(p0, p1, h_env, c1, c2, b_env, ln_g, ln_b, bs=2000):
    return pl.pallas_call(
        _final_body,
        grid=(N // bs,),
        in_specs=[
            pl.BlockSpec((bs, 64), lambda i: (i, 0)),
            pl.BlockSpec((bs, 64), lambda i: (i, 0)),
            pl.BlockSpec((bs, D), lambda i: (i, 0)),
            pl.BlockSpec((bs, 1), lambda i: (i, 0)),
            pl.BlockSpec((bs, 1), lambda i: (i, 0)),
            pl.BlockSpec((1, D), lambda i: (0, 0)),
            pl.BlockSpec((1, D), lambda i: (0, 0)),
            pl.BlockSpec((1, D), lambda i: (0, 0)),
        ],
        out_specs=pl.BlockSpec((bs, D), lambda i: (i, 0)),
        out_shape=jax.ShapeDtypeStruct((N, D), jnp.float32),
    )(p0, p1, h_env, c1.reshape(N, 1), c2.reshape(N, 1),
      b_env.reshape(1, D), ln_g.reshape(1, D), ln_b.reshape(1, D))


def _gumbel_hard0(logits, g):
    y = jax.nn.softmax((logits + g) / TEMP, axis=-1)
    idx = jnp.argmax(y, axis=-1)
    y_hard = jax.nn.one_hot(idx, 2, dtype=y.dtype)
    return ((y_hard - y) + y)[:, 0]


def kernel(x, edge_index, W_in, b_in, W_out, b_out, W_env, b_env,
           ln_in_g, ln_in_b, ln_out_g, ln_out_b):
    src, dst = edge_index[0], edge_index[1]
    pad = jnp.zeros((E_PAD - E,), dtype=jnp.int32)
    pad_d = N + jnp.arange(E_PAD - E, dtype=jnp.int32) % (N_ACC - N)
    src3 = jnp.concatenate([src, pad]).reshape(NW, NB, 128)
    dst3 = jnp.concatenate([dst, pad_d]).reshape(NW, NB, 128)

    W4 = jnp.concatenate([W_in, W_out], axis=1)
    b4 = jnp.concatenate([b_in, b_out])
    h4, h_env = _front(x, ln_in_g, ln_in_b, W4, W_env)

    zrow8 = jnp.zeros((RPT, 8), jnp.float32)

    def to8(t):
        return jnp.concatenate(
            [t, jnp.zeros((N, 8 - t.shape[1]), jnp.float32)], axis=1)

    ones_t = to8(jnp.ones((N, 1), jnp.float32))
    cnt = _scatter8(src3, dst3, ones_t, zrow8)
    cnt = cnt[0, :N, 0] + cnt[1, :N, 0]
    dinv_u = 1.0 / jnp.sqrt(cnt + 1.0)

    h4s = to8(dinv_u[:, None] * h4)
    pre4 = _scatter8(src3, dst3, h4s, zrow8)
    pre4 = pre4[0, :N, :4] + pre4[1, :N, :4]
    logits4 = dinv_u[:, None] * pre4 + (dinv_u ** 2)[:, None] * h4 + b4

    kg = jax.random.key(42)
    u1 = jax.random.uniform(jax.random.fold_in(kg, 0), (N, 2),
                            minval=1e-6, maxval=1.0)
    u2 = jax.random.uniform(jax.random.fold_in(kg, 1), (N, 2),
                            minval=1e-6, maxval=1.0)
    g1 = -jnp.log(-jnp.log(u1))
    g2 = -jnp.log(-jnp.log(u2))
    in_val = _gumbel_hard0(logits4[:, :2], g1)
    out_val = _gumbel_hard0(logits4[:, 2:], g2)

    zpad1 = jnp.zeros((N_ACC - N,), jnp.float32)
    src1w = jnp.concatenate([src, pad]).reshape(NW, EPW)
    dst1w = jnp.concatenate([dst, pad_d]).reshape(NW, EPW)
    z80 = jnp.zeros((80, 128), jnp.float32)
    sout_p = _gate_sout(src1w, dst1w,
                        jnp.concatenate([out_val, zpad1]), z80)
    s_out = (sout_p[0] + sout_p[1]).reshape(N_ACC)[:N]
    deg_w = in_val * s_out + 1.0
    dinv_w = 1.0 / jnp.sqrt(deg_w)

    hh = _scale_rows_split(out_val * dinv_w, h_env)
    zrow64 = jnp.zeros((N_ACC, 64), jnp.float32)
    src3s = jnp.concatenate([src, pad]).reshape(NS, NBT, 128)
    dst3s = jnp.concatenate([dst, pad_d]).reshape(NS, NBT, 128)
    pre = _scatter_split(src3s, dst3s, hh, zrow64)

    c1 = dinv_w * in_val
    c2 = dinv_w ** 2
    return _final(pre[0, :N], pre[1, :N], h_env, c1, c2,
                  b_env, ln_out_g, ln_out_b)


# speedup vs baseline: 1.2790x; 1.1165x over previous
import functools

import jax
import jax.numpy as jnp
from jax import lax
from jax.experimental import pallas as pl
from jax.experimental.pallas import tpu as pltpu
from jax.experimental.pallas import tpu_sc as plsc

N = 10000
E = 320000
D = 128
TEMP = 0.5

NC, NS, L = 2, 16, 16
NW = NC * NS
NB = 80
E_PAD = NW * NB * 128
N_ACC = 10240
RPT = N_ACC // NS
EPW = E_PAD // NW


def _make_scatter(Dw, C):
    NBC = NB // C
    mesh = plsc.VectorSubcoreMesh(core_axis_name="c", subcore_axis_name="s")

    @functools.partial(
        pl.kernel,
        mesh=mesh,
        compiler_params=pltpu.CompilerParams(use_tc_tiling_on_sc=False),
        out_type=jax.ShapeDtypeStruct((NC, N_ACC, Dw), jnp.float32),
        scratch_types=[
            pltpu.VMEM((NB, 128), jnp.int32),
            pltpu.VMEM((NB, 128), jnp.int32),
            pltpu.VMEM((C, 128, Dw), jnp.float32),
            pltpu.VMEM_SHARED((N_ACC, Dw), jnp.float32),
            pltpu.SemaphoreType.DMA,
            pltpu.SemaphoreType.DMA,
        ],
    )
    def k(src_hbm, dst_hbm, table_hbm, zrow_hbm, out_hbm,
          src_v, dst_v, rows_v, acc, sem_g, sem_s):
        cid = lax.axis_index("c")
        sid = lax.axis_index("s")
        wid = sid * NC + cid
        pltpu.sync_copy(zrow_hbm, acc.at[pl.ds(sid * RPT, RPT)])
        pltpu.sync_copy(src_hbm.at[wid], src_v)
        pltpu.sync_copy(dst_hbm.at[wid], dst_v)
        plsc.subcore_barrier()

        def chunk(i, carry):
            gh = [pltpu.async_copy(table_hbm.at[src_v.at[i * C + b]],
                                   rows_v.at[b], sem_g)
                  for b in range(C)]
            sh = []
            for b in range(C):
                gh[b].wait()
                sh.append(pltpu.async_copy(rows_v.at[b],
                                           acc.at[dst_v.at[i * C + b]],
                                           sem_s, add=True))
            for b in range(C):
                sh[b].wait()
            return carry

        lax.fori_loop(0, NBC, chunk, 0)
        plsc.subcore_barrier()
        pltpu.sync_copy(acc.at[pl.ds(sid * RPT, RPT)],
                        out_hbm.at[cid, pl.ds(sid * RPT, RPT)])

    return k


_scatter8 = _make_scatter(8, 16)

CS = 5
_mesh_split = plsc.VectorSubcoreMesh(core_axis_name="c", subcore_axis_name="s")


@functools.partial(
    pl.kernel,
    mesh=_mesh_split,
    compiler_params=pltpu.CompilerParams(use_tc_tiling_on_sc=False,
                                         needs_layout_passes=False),
    out_type=jax.ShapeDtypeStruct((NC, 80, 128), jnp.float32),
    scratch_types=[
        pltpu.VMEM((EPW,), jnp.int32),
        pltpu.VMEM((EPW,), jnp.int32),
        pltpu.VMEM((N_ACC,), jnp.float32),
        pltpu.VMEM((80, 128), jnp.float32),
        pltpu.VMEM((1, 80), jnp.int32),
        pltpu.VMEM_SHARED((80, 128), jnp.float32),
    ],
)
def _gate_sout(src_hbm, dst_hbm, outval_hbm, z80_hbm, sout_hbm,
               src_v, dst_v, outv_v, souts_v, idx80_v, sacc):
    cid = lax.axis_index("c")
    sid = lax.axis_index("s")
    wid = sid * NC + cid
    pltpu.sync_copy(src_hbm.at[wid], src_v)
    pltpu.sync_copy(dst_hbm.at[wid], dst_v)
    pltpu.sync_copy(outval_hbm, outv_v)
    pltpu.sync_copy(z80_hbm, souts_v)
    pltpu.sync_copy(z80_hbm.at[pl.ds(sid * 5, 5)], sacc.at[pl.ds(sid * 5, 5)])
    for kk in range(5):
        idx80_v[0, pl.ds(kk * 16, 16)] = (
            jnp.arange(16, dtype=jnp.int32) + 16 * kk)

    def body(i, c):
        s16 = src_v[pl.ds(i * 16, 16)]
        d16 = dst_v[pl.ds(i * 16, 16)]
        ov = plsc.load_gather(outv_v, [s16])
        rr = jnp.right_shift(d16, 7)
        cc = jnp.bitwise_and(d16, 127)
        plsc.addupdate_scatter(souts_v, [rr, cc], ov)
        return c

    lax.fori_loop(0, EPW // 16, body, 0)
    plsc.subcore_barrier()
    pltpu.sync_copy(souts_v, sacc.at[idx80_v.at[0]], add=True)
    plsc.subcore_barrier()
    pltpu.sync_copy(sacc.at[pl.ds(sid * 5, 5)],
                    sout_hbm.at[cid, pl.ds(sid * 5, 5)])


@functools.partial(
    pl.kernel,
    mesh=_mesh_split,
    compiler_params=pltpu.CompilerParams(use_tc_tiling_on_sc=False,
                                         needs_layout_passes=False),
    out_type=jax.ShapeDtypeStruct((NC, 80, 128), jnp.float32),
    scratch_types=[
        pltpu.VMEM((EPW,), jnp.int32),
        pltpu.VMEM((80, 128), jnp.float32),
        pltpu.VMEM((1, 80), jnp.int32),
        pltpu.VMEM_SHARED((80, 128), jnp.float32),
    ],
)
def _hist_reg(dst_hbm, z80_hbm, out_hbm, dst_v, priv_v, idx80_v, sacc):
    cid = lax.axis_index("c")
    sid = lax.axis_index("s")
    wid = sid * NC + cid
    pltpu.sync_copy(dst_hbm.at[wid], dst_v)
    pltpu.sync_copy(z80_hbm, priv_v)
    pltpu.sync_copy(z80_hbm.at[pl.ds(sid * 5, 5)], sacc.at[pl.ds(sid * 5, 5)])
    for kk in range(5):
        idx80_v[0, pl.ds(kk * 16, 16)] = (
            jnp.arange(16, dtype=jnp.int32) + 16 * kk)

    def body(i, c):
        d16 = dst_v[pl.ds(i * 16, 16)]
        rr = jnp.right_shift(d16, 7)
        cc = jnp.bitwise_and(d16, 127)
        plsc.addupdate_scatter(priv_v, [rr, cc], jnp.ones((16,), jnp.float32))
        return c

    lax.fori_loop(0, EPW // 16, body, 0)
    plsc.subcore_barrier()
    pltpu.sync_copy(priv_v, sacc.at[idx80_v.at[0]], add=True)
    plsc.subcore_barrier()
    pltpu.sync_copy(sacc.at[pl.ds(sid * 5, 5)],
                    out_hbm.at[cid, pl.ds(sid * 5, 5)])


EPW2 = E_PAD // NS


@functools.partial(
    pl.kernel,
    mesh=_mesh_split,
    compiler_params=pltpu.CompilerParams(use_tc_tiling_on_sc=False,
                                         needs_layout_passes=False),
    out_type=jax.ShapeDtypeStruct((NC, 2, 80, 128), jnp.float32),
    scratch_types=[
        pltpu.VMEM((EPW2 // 4,), jnp.int32),
        pltpu.VMEM((EPW2 // 4,), jnp.int32),
        pltpu.VMEM((N, 2), jnp.float32),
        pltpu.VMEM((2, 80, 128), jnp.float32),
        pltpu.VMEM((1, 80), jnp.int32),
        pltpu.VMEM_SHARED((2, 80, 128), jnp.float32),
    ],
)
def _pre4_reg(src_hbm, dst_hbm, h4s_hbm, z280_hbm, out_hbm,
              src_v, dst_v, h4s_v, priv_v, idx80_v, sacc):
    cid = lax.axis_index("c")
    sid = lax.axis_index("s")
    pltpu.sync_copy(h4s_hbm.at[cid], h4s_v)
    pltpu.sync_copy(z280_hbm, priv_v)
    pltpu.sync_copy(z280_hbm.at[0, pl.ds(sid * 5, 5)],
                    sacc.at[0, pl.ds(sid * 5, 5)])
    pltpu.sync_copy(z280_hbm.at[1, pl.ds(sid * 5, 5)],
                    sacc.at[1, pl.ds(sid * 5, 5)])
    for kk in range(5):
        idx80_v[0, pl.ds(kk * 16, 16)] = (
            jnp.arange(16, dtype=jnp.int32) + 16 * kk)

    def body(i, c):
        s16 = src_v[pl.ds(i * 16, 16)]
        d16 = dst_v[pl.ds(i * 16, 16)]
        rr = jnp.right_shift(d16, 7)
        cc = jnp.bitwise_and(d16, 127)
        for p in range(2):
            vp = plsc.load_gather(h4s_v,
                                  [s16, jnp.full((16,), p, jnp.int32)])
            plsc.addupdate_scatter(priv_v.at[p], [rr, cc], vp)
        return c

    for q in range(4):
        pltpu.sync_copy(src_hbm.at[sid, pl.ds(q * (EPW2 // 4), EPW2 // 4)],
                        src_v)
        pltpu.sync_copy(dst_hbm.at[sid, pl.ds(q * (EPW2 // 4), EPW2 // 4)],
                        dst_v)
        lax.fori_loop(0, EPW2 // 64, body, 0)
    plsc.subcore_barrier()
    for p in range(2):
        pltpu.sync_copy(priv_v.at[p], sacc.at[p].at[idx80_v.at[0]], add=True)
    plsc.subcore_barrier()
    for p in range(2):
        pltpu.sync_copy(sacc.at[p, pl.ds(sid * 5, 5)],
                        out_hbm.at[cid, p, pl.ds(sid * 5, 5)])


NBT = E_PAD // (NS * 128)


@functools.partial(
    pl.kernel,
    mesh=_mesh_split,
    compiler_params=pltpu.CompilerParams(use_tc_tiling_on_sc=False),
    out_type=jax.ShapeDtypeStruct((NC, N_ACC, 64), jnp.float32),
    scratch_types=[
        pltpu.VMEM((NBT, 128), jnp.int32),
        pltpu.VMEM((NBT, 128), jnp.int32),
        pltpu.VMEM((CS, 128, 64), jnp.float32),
        pltpu.VMEM_SHARED((N_ACC, 64), jnp.float32),
        pltpu.SemaphoreType.DMA,
        pltpu.SemaphoreType.DMA,
    ],
)
def _scatter_split(src_hbm, dst_hbm, table_hbm, zrow_hbm, out_hbm,
                   src_v, dst_v, rows_v, acc, sem_g, sem_s):
    cid = lax.axis_index("c")
    sid = lax.axis_index("s")
    pltpu.sync_copy(zrow_hbm.at[pl.ds(sid * RPT, RPT)],
                    acc.at[pl.ds(sid * RPT, RPT)])
    pltpu.sync_copy(src_hbm.at[sid], src_v)
    pltpu.sync_copy(dst_hbm.at[sid], dst_v)
    plsc.subcore_barrier()

    def chunk(i, carry):
        gh = [pltpu.async_copy(table_hbm.at[cid].at[src_v.at[i * CS + b]],
                               rows_v.at[b], sem_g)
              for b in range(CS)]
        sh = []
        for b in range(CS):
            gh[b].wait()
            sh.append(pltpu.async_copy(rows_v.at[b],
                                       acc.at[dst_v.at[i * CS + b]],
                                       sem_s, add=True))
        for b in range(CS):
            sh[b].wait()
        return carry

    lax.fori_loop(0, NBT // CS, chunk, 0)
    plsc.subcore_barrier()
    pltpu.sync_copy(acc.at[pl.ds(sid * RPT, RPT)],
                    out_hbm.at[cid, pl.ds(sid * RPT, RPT)])


def _front_body(x_ref, g_ref, b_ref, w4_ref, wenv_ref, h4_ref, henv_ref):
    x = x_ref[...]
    mu = jnp.mean(x, axis=-1, keepdims=True)
    var = jnp.mean((x - mu) ** 2, axis=-1, keepdims=True)
    xn = (x - mu) / jnp.sqrt(var + 1e-5) * g_ref[...] + b_ref[...]
    h4_ref[...] = jnp.dot(xn, w4_ref[...], preferred_element_type=jnp.float32)
    henv_ref[...] = jnp.dot(xn, wenv_ref[...], preferred_element_type=jnp.float32)


def _front(x, ln_g, ln_b, W4, W_env, bs=2000):
    grid = (N // bs,)
    return pl.pallas_call(
        _front_body,
        grid=grid,
        in_specs=[
            pl.BlockSpec((bs, D), lambda i: (i, 0)),
            pl.BlockSpec((1, D), lambda i: (0, 0)),
            pl.BlockSpec((1, D), lambda i: (0, 0)),
            pl.BlockSpec((D, 4), lambda i: (0, 0)),
            pl.BlockSpec((D, D), lambda i: (0, 0)),
        ],
        out_specs=[
            pl.BlockSpec((bs, 4), lambda i: (i, 0)),
            pl.BlockSpec((bs, D), lambda i: (i, 0)),
        ],
        out_shape=[
            jax.ShapeDtypeStruct((N, 4), jnp.float32),
            jax.ShapeDtypeStruct((N, D), jnp.float32),
        ],
    )(x, ln_g.reshape(1, D), ln_b.reshape(1, D), W4, W_env)


def _scale_body(a_ref, h_ref, o_ref):
    hh = a_ref[...] * h_ref[...]
    o_ref[0] = hh[:, :64]
    o_ref[1] = hh[:, 64:]


def _scale_rows_split(a, h, bs=2000):
    return pl.pallas_call(
        _scale_body,
        grid=(N // bs,),
        in_specs=[
            pl.BlockSpec((bs, 1), lambda i: (i, 0)),
            pl.BlockSpec((bs, D), lambda i: (i, 0)),
        ],
        out_specs=pl.BlockSpec((NC, bs, 64), lambda i: (0, i, 0)),
        out_shape=jax.ShapeDtypeStruct((NC, N, 64), jnp.float32),
    )(a.reshape(N, 1), h)


def _final_body(p0_ref, p1_ref, henv_ref, c1_ref, c2_ref, be_ref,
                g_ref, b_ref, o_ref):
    pre = jnp.concatenate([p0_ref[...], p1_ref[...]], axis=1)
    o = (c1_ref[...] * pre
         + c2_ref[...] * henv_ref[...] + be_ref[...])
    mu = jnp.mean(o, axis=-1, keepdims=True)
    var = jnp.mean((o - mu) ** 2, axis=-1, keepdims=True)
    o_ref[...] = (o - mu) / jnp.sqrt(var + 1e-5) * g_ref[...] + b_ref[...]


def _final(p0, p1, h_env, c1, c2, b_env, ln_g, ln_b, bs=2000):
    return pl.pallas_call(
        _final_body,
        grid=(N // bs,),
        in_specs=[
            pl.BlockSpec((bs, 64), lambda i: (i, 0)),
            pl.BlockSpec((bs, 64), lambda i: (i, 0)),
            pl.BlockSpec((bs, D), lambda i: (i, 0)),
            pl.BlockSpec((bs, 1), lambda i: (i, 0)),
            pl.BlockSpec((bs, 1), lambda i: (i, 0)),
            pl.BlockSpec((1, D), lambda i: (0, 0)),
            pl.BlockSpec((1, D), lambda i: (0, 0)),
            pl.BlockSpec((1, D), lambda i: (0, 0)),
        ],
        out_specs=pl.BlockSpec((bs, D), lambda i: (i, 0)),
        out_shape=jax.ShapeDtypeStruct((N, D), jnp.float32),
    )(p0, p1, h_env, c1.reshape(N, 1), c2.reshape(N, 1),
      b_env.reshape(1, D), ln_g.reshape(1, D), ln_b.reshape(1, D))


def _gumbel_hard0(logits, g):
    y = jax.nn.softmax((logits + g) / TEMP, axis=-1)
    idx = jnp.argmax(y, axis=-1)
    y_hard = jax.nn.one_hot(idx, 2, dtype=y.dtype)
    return ((y_hard - y) + y)[:, 0]


def kernel(x, edge_index, W_in, b_in, W_out, b_out, W_env, b_env,
           ln_in_g, ln_in_b, ln_out_g, ln_out_b):
    src, dst = edge_index[0], edge_index[1]
    pad = jnp.zeros((E_PAD - E,), dtype=jnp.int32)
    pad_d = N + jnp.arange(E_PAD - E, dtype=jnp.int32) % (N_ACC - N)
    src3 = jnp.concatenate([src, pad]).reshape(NW, NB, 128)
    dst3 = jnp.concatenate([dst, pad_d]).reshape(NW, NB, 128)

    W4 = jnp.concatenate([W_in, W_out], axis=1)
    b4 = jnp.concatenate([b_in, b_out])
    h4, h_env = _front(x, ln_in_g, ln_in_b, W4, W_env)

    z80g = jnp.zeros((80, 128), jnp.float32)
    src1wg = jnp.concatenate([src, pad]).reshape(NW, EPW)
    dst1wg = jnp.concatenate([dst, pad_d]).reshape(NW, EPW)

    cnt_p = _hist_reg(dst1wg, z80g)
    cnt = (cnt_p[0] + cnt_p[1]).reshape(N_ACC)[:N]
    dinv_u = 1.0 / jnp.sqrt(cnt + 1.0)

    h4s = dinv_u[:, None] * h4
    h4s2 = jnp.stack([h4s[:, :2], h4s[:, 2:]])
    z280 = jnp.zeros((2, 80, 128), jnp.float32)
    src1t = jnp.concatenate([src, pad]).reshape(NS, EPW2)
    dst1t = jnp.concatenate([dst, pad_d]).reshape(NS, EPW2)
    pre4_p = _pre4_reg(src1t, dst1t, h4s2, z280)
    pre4 = jnp.concatenate([pre4_p[0], pre4_p[1]]).reshape(4, N_ACC)[:, :N].T
    logits4 = dinv_u[:, None] * pre4 + (dinv_u ** 2)[:, None] * h4 + b4

    kg = jax.random.key(42)
    u1 = jax.random.uniform(jax.random.fold_in(kg, 0), (N, 2),
                            minval=1e-6, maxval=1.0)
    u2 = jax.random.uniform(jax.random.fold_in(kg, 1), (N, 2),
                            minval=1e-6, maxval=1.0)
    g1 = -jnp.log(-jnp.log(u1))
    g2 = -jnp.log(-jnp.log(u2))
    in_val = _gumbel_hard0(logits4[:, :2], g1)
    out_val = _gumbel_hard0(logits4[:, 2:], g2)

    zpad1 = jnp.zeros((N_ACC - N,), jnp.float32)
    sout_p = _gate_sout(src1wg, dst1wg,
                        jnp.concatenate([out_val, zpad1]), z80g)
    s_out = (sout_p[0] + sout_p[1]).reshape(N_ACC)[:N]
    deg_w = in_val * s_out + 1.0
    dinv_w = 1.0 / jnp.sqrt(deg_w)

    hh = _scale_rows_split(out_val * dinv_w, h_env)
    zrow64 = jnp.zeros((N_ACC, 64), jnp.float32)
    src3s = jnp.concatenate([src, pad]).reshape(NS, NBT, 128)
    dst3s = jnp.concatenate([dst, pad_d]).reshape(NS, NBT, 128)
    pre = _scatter_split(src3s, dst3s, hh, zrow64)

    c1 = dinv_w * in_val
    c2 = dinv_w ** 2
    return _final(pre[0, :N], pre[1, :N], h_env, c1, c2,
                  b_env, ln_out_g, ln_out_b)
